# baseline (speedup 1.0000x reference)
import functools

import jax
import jax.numpy as jnp
from jax import lax
from jax.experimental import pallas as pl
from jax.experimental.pallas import tpu as pltpu
from jax.experimental.pallas import tpu_sc as plsc

_UNIT = 512


def _build(B0, B1, V, D, n_workers, num_cores):
    mesh = plsc.VectorSubcoreMesh(core_axis_name="c", subcore_axis_name="s")
    n_sub = _UNIT // 16
    assert B0 % (_UNIT * n_workers) == 0 or B0 == _UNIT * n_workers
    assert B1 % 2 == 0

    @functools.partial(
        pl.kernel,
        mesh=mesh,
        out_type=jax.ShapeDtypeStruct((B1 * D, B0), jnp.float32),
        scratch_types=[
            pltpu.VMEM((_UNIT, B1), jnp.int32),
            pltpu.VMEM((2, _UNIT), jnp.int32),
            pltpu.VMEM((2, _UNIT, D), jnp.float32),
            pltpu.VMEM((2, D, _UNIT), jnp.float32),
        ]
        + [pltpu.SemaphoreType.DMA] * 5,
        compiler_params=pltpu.CompilerParams(
            use_tc_tiling_on_sc=False, needs_layout_passes=False
        ),
    )
    def k(x_hbm, table_hbm, out_hbm, xblk_v, idx_v, rows_v, tr_v, *sems):
        xsem = sems[0]
        gsem = sems[1:3]
        ssem = sems[3:5]
        wid = lax.axis_index("s") * num_cores + lax.axis_index("c")
        i0 = wid * _UNIT

        iota = lax.iota(jnp.int32, 16)
        zeros16 = jnp.zeros((16,), jnp.int32)
        cvecs = [iota + (h * 16) for h in range(D // 16)]
        rowvecs = [iota + (s * 16) for s in range(n_sub)]

        def build_idx(j, b):
            jsplat = zeros16 + j
            for s in range(n_sub):
                v = plsc.load_gather(xblk_v, [rowvecs[s], jsplat])
                idx_v[b, pl.ds(s * 16, 16)] = v * 4

        def start_gather(b):
            pltpu.async_copy(table_hbm.at[idx_v.at[b]], rows_v.at[b], gsem[b])

        def wait_gather(b):
            pltpu.make_async_copy(
                table_hbm.at[pl.ds(0, _UNIT)], rows_v.at[b], gsem[b]
            ).wait()

        def start_store(j, b):
            pltpu.async_copy(
                tr_v.at[b],
                out_hbm.at[pl.ds(j * D, D), pl.ds(i0, _UNIT)],
                ssem[b],
            )

        def wait_store(b):
            pltpu.make_async_copy(
                out_hbm.at[pl.ds(0, D), pl.ds(0, _UNIT)], tr_v.at[b], ssem[b]
            ).wait()

        def transpose(b):
            def dbody(b4, carry):
                base = b4 * 4
                for db in range(4):
                    bs = zeros16 + (base + db)
                    for h in range(D // 16):
                        rvec = (bs + cvecs[h]) & (_UNIT - 1)
                        v = plsc.load_gather(rows_v.at[b], [rvec, cvecs[h]])
                        plsc.store_scatter(tr_v.at[b], [cvecs[h], rvec], v)
                return carry

            lax.fori_loop(0, _UNIT // 4, dbody, 0)

        def finish(j, b, guard):
            wait_gather(b)

            @pl.when(guard)
            def _():
                wait_store(b)

            transpose(b)
            start_store(j, b)

        pltpu.async_copy(x_hbm.at[pl.ds(i0, _UNIT), :], xblk_v, xsem)
        pltpu.make_async_copy(
            x_hbm.at[pl.ds(0, _UNIT), :], xblk_v, xsem
        ).wait()
        build_idx(0, 0)
        start_gather(0)

        def body(g, carry):
            j = 2 * g
            build_idx(j + 1, 1)
            start_gather(1)
            finish(j, 0, g >= 1)
            build_idx(j + 2, 0)
            start_gather(0)
            finish(j + 1, 1, g >= 1)
            return carry

        lax.fori_loop(0, (B1 - 2) // 2, body, 0)

        build_idx(B1 - 1, 1)
        start_gather(1)
        wait_gather(0)
        wait_store(0)
        transpose(0)
        start_store(B1 - 2, 0)
        wait_gather(1)
        wait_store(1)
        transpose(1)
        start_store(B1 - 1, 1)
        wait_store(0)
        wait_store(1)

    return k


def kernel(x, table):
    B0, B1 = x.shape
    V, D = table.shape

    info = plsc.get_sparse_core_info()
    n_workers = info.num_cores * info.num_subcores
    table4 = jnp.pad(table, ((0, 0), (0, 128 - D))).reshape(V * (128 // D), D)
    k = _build(B0, B1, V, D, n_workers, info.num_cores)
    out2d = k(x, table4)
    return out2d.reshape(B1, D, B0).transpose(2, 0, 1)


# speedup vs baseline: 1.0053x; 1.0053x over previous
import functools

import jax
import jax.numpy as jnp
from jax import lax
from jax.experimental import pallas as pl
from jax.experimental.pallas import tpu as pltpu
from jax.experimental.pallas import tpu_sc as plsc

_UNIT = 512


def _build(B0, B1, V, D, n_workers, num_cores):
    mesh = plsc.VectorSubcoreMesh(core_axis_name="c", subcore_axis_name="s")
    n_sub = _UNIT // 16
    assert B0 % (_UNIT * n_workers) == 0 or B0 == _UNIT * n_workers
    assert B1 % 2 == 0

    @functools.partial(
        pl.kernel,
        mesh=mesh,
        out_type=jax.ShapeDtypeStruct((B1 * D, B0), jnp.float32),
        scratch_types=[
            pltpu.VMEM((_UNIT, B1), jnp.int32),
            pltpu.VMEM((2, _UNIT), jnp.int32),
            pltpu.VMEM((2, _UNIT, D), jnp.float32),
            pltpu.VMEM((2, D, _UNIT), jnp.float32),
        ]
        + [pltpu.SemaphoreType.DMA] * 5,
        compiler_params=pltpu.CompilerParams(
            use_tc_tiling_on_sc=False, needs_layout_passes=False
        ),
    )
    def k(x_hbm, table_hbm, out_hbm, xblk_v, idx_v, rows_v, tr_v, *sems):
        xsem = sems[0]
        gsem = sems[1:3]
        ssem = sems[3:5]
        wid = lax.axis_index("s") * num_cores + lax.axis_index("c")
        i0 = wid * _UNIT

        iota = lax.iota(jnp.int32, 16)
        zeros16 = jnp.zeros((16,), jnp.int32)
        cvecs = [iota + (h * 16) for h in range(D // 16)]
        rowvecs = [iota + (s * 16) for s in range(n_sub)]

        def build_idx(j, b):
            jsplat = zeros16 + j
            for s in range(n_sub):
                v = plsc.load_gather(xblk_v, [rowvecs[s], jsplat])
                idx_v[b, pl.ds(s * 16, 16)] = v * 4

        def start_gather(b):
            pltpu.async_copy(table_hbm.at[idx_v.at[b]], rows_v.at[b], gsem[b])

        def wait_gather(b):
            pltpu.make_async_copy(
                table_hbm.at[pl.ds(0, _UNIT)], rows_v.at[b], gsem[b]
            ).wait()

        def start_store(j, b):
            pltpu.async_copy(
                tr_v.at[b],
                out_hbm.at[pl.ds(j * D, D), pl.ds(i0, _UNIT)],
                ssem[b],
            )

        def wait_store(b):
            pltpu.make_async_copy(
                out_hbm.at[pl.ds(0, D), pl.ds(0, _UNIT)], tr_v.at[b], ssem[b]
            ).wait()

        def transpose(b):
            def dbody(b4, carry):
                base = b4 * 8
                for db in range(8):
                    bs = zeros16 + (base + db)
                    for h in range(D // 16):
                        rvec = (bs + cvecs[h]) & (_UNIT - 1)
                        v = plsc.load_gather(rows_v.at[b], [rvec, cvecs[h]])
                        plsc.store_scatter(tr_v.at[b], [cvecs[h], rvec], v)
                return carry

            lax.fori_loop(0, _UNIT // 8, dbody, 0)

        def finish(j, b, guard):
            wait_gather(b)

            @pl.when(guard)
            def _():
                wait_store(b)

            transpose(b)
            start_store(j, b)

        pltpu.async_copy(x_hbm.at[pl.ds(i0, _UNIT), :], xblk_v, xsem)
        pltpu.make_async_copy(
            x_hbm.at[pl.ds(0, _UNIT), :], xblk_v, xsem
        ).wait()
        build_idx(0, 0)
        start_gather(0)

        def body(g, carry):
            j = 2 * g
            build_idx(j + 1, 1)
            start_gather(1)
            finish(j, 0, g >= 1)
            build_idx(j + 2, 0)
            start_gather(0)
            finish(j + 1, 1, g >= 1)
            return carry

        lax.fori_loop(0, (B1 - 2) // 2, body, 0)

        build_idx(B1 - 1, 1)
        start_gather(1)
        wait_gather(0)
        wait_store(0)
        transpose(0)
        start_store(B1 - 2, 0)
        wait_gather(1)
        wait_store(1)
        transpose(1)
        start_store(B1 - 1, 1)
        wait_store(0)
        wait_store(1)

    return k


def kernel(x, table):
    B0, B1 = x.shape
    V, D = table.shape

    info = plsc.get_sparse_core_info()
    n_workers = info.num_cores * info.num_subcores
    table4 = jnp.pad(table, ((0, 0), (0, 128 - D))).reshape(V * (128 // D), D)
    k = _build(B0, B1, V, D, n_workers, info.num_cores)
    out2d = k(x, table4)
    return out2d.reshape(B1, D, B0).transpose(2, 0, 1)
